# baseline (speedup 1.0000x reference)
import jax
import jax.numpy as jnp
from jax import lax
from jax.experimental import pallas as pl
from jax.experimental.pallas import tpu as pltpu
from jax.experimental.pallas import tpu_sc as plsc

_E = 16
_D = 768
_DFF = 2048
_T = 2048
_BM = 128
_NTILES = 32
_TPAD = _NTILES * _BM
_NBUF = 3

_SCA_W = 16
_SCA_TOK = _T // _SCA_W
_SCB_W = 32
_SCB_TOK = _T // _SCB_W


def _router_body(x_ref, wr_ref, out_ref):
    out_ref[...] = jax.lax.dot_general(
        x_ref[...], wr_ref[...], (((1,), (1,)), ((), ())),
        preferred_element_type=jnp.float32)


def _router_logits(x, w_router):
    return pl.pallas_call(
        _router_body,
        out_shape=jax.ShapeDtypeStruct((_T, _E), jnp.float32),
    )(x, w_router)


def _sca1_body(logits_hbm, eid_hbm, rank_hbm, wden_hbm, cnt_hbm,
               lv_ref, eid_ref, rank_ref, wv_ref, cnt_ref):
    wid = lax.axis_index("s") * 2 + lax.axis_index("c")
    base = wid * _SCB_TOK
    pltpu.sync_copy(logits_hbm.at[pl.ds(base, _SCB_TOK)], lv_ref)

    lane = lax.iota(jnp.int32, 16)
    cnt = jnp.zeros((16,), jnp.int32)
    for g in range(_SCB_TOK // 16):
        acc_e = jnp.zeros((16,), jnp.int32)
        acc_r = jnp.zeros((16,), jnp.int32)
        acc_w = jnp.zeros((16,), jnp.float32)
        for j in range(16):
            lv = lv_ref[g * 16 + j, :]
            m = jnp.max(lv)
            e_splat = plsc.all_reduce_ffs(lv == m)
            wgt = jnp.sum(jnp.exp(lv - m))
            onehot = (lane == e_splat).astype(jnp.int32)
            rank = jnp.sum(cnt * onehot)
            cnt = cnt + onehot
            sel = lane == j
            acc_e = jnp.where(sel, e_splat, acc_e)
            acc_r = jnp.where(sel, rank, acc_r)
            acc_w = jnp.where(sel, wgt, acc_w)
        eid_ref[pl.ds(g * 16, 16)] = acc_e
        rank_ref[pl.ds(g * 16, 16)] = acc_r
        wv_ref[pl.ds(g * 16, 16)] = acc_w

    cnt_ref[...] = cnt
    pltpu.sync_copy(eid_ref, eid_hbm.at[pl.ds(base, _SCB_TOK)])
    pltpu.sync_copy(rank_ref, rank_hbm.at[pl.ds(base, _SCB_TOK)])
    pltpu.sync_copy(wv_ref, wden_hbm.at[pl.ds(base, _SCB_TOK)])
    pltpu.sync_copy(cnt_ref, cnt_hbm.at[wid])


def _sc_route(logits):
    f = pl.kernel(
        _sca1_body,
        out_type=(
            jax.ShapeDtypeStruct((_T,), jnp.int32),
            jax.ShapeDtypeStruct((_T,), jnp.int32),
            jax.ShapeDtypeStruct((_T,), jnp.float32),
            jax.ShapeDtypeStruct((_SCB_W, _E), jnp.int32),
        ),
        mesh=plsc.VectorSubcoreMesh(
            core_axis_name="c", subcore_axis_name="s"),
        compiler_params=pltpu.CompilerParams(needs_layout_passes=False),
        scratch_types=[
            pltpu.VMEM((_SCB_TOK, _E), jnp.float32),
            pltpu.VMEM((_SCB_TOK,), jnp.int32),
            pltpu.VMEM((_SCB_TOK,), jnp.int32),
            pltpu.VMEM((_SCB_TOK,), jnp.float32),
            pltpu.VMEM((_E,), jnp.int32),
        ],
    )
    return f(logits)


def _sca2_body(x_hbm, eid_hbm, rank_hbm, off_hbm,
               xpad_hbm, pos_hbm,
               eid_ref, rank_ref, posv_ref, off_ref, xrow_ref,
               sem):
    wid = lax.axis_index("s") * 2 + lax.axis_index("c")
    base = wid * _SCB_TOK
    pltpu.sync_copy(eid_hbm.at[pl.ds(base, _SCB_TOK)], eid_ref)
    pltpu.sync_copy(rank_hbm.at[pl.ds(base, _SCB_TOK)], rank_ref)
    pltpu.sync_copy(off_hbm.at[wid], off_ref)

    for g in range(_SCB_TOK // 16):
        ev = eid_ref[pl.ds(g * 16, 16)]
        offg = plsc.load_gather(off_ref, [ev])
        posv_ref[pl.ds(g * 16, 16)] = offg + rank_ref[pl.ds(g * 16, 16)]

    pltpu.sync_copy(x_hbm.at[pl.ds(base, _SCB_TOK)], xrow_ref)
    cp = pltpu.make_async_copy(xrow_ref, xpad_hbm.at[posv_ref], sem)
    cp.start()
    cp.wait()
    pltpu.sync_copy(posv_ref, pos_hbm.at[pl.ds(base, _SCB_TOK)])


def _sc_dispatch(x, eid, rank, off):
    f = pl.kernel(
        _sca2_body,
        out_type=(
            jax.ShapeDtypeStruct((_TPAD, _D), jnp.float32),
            jax.ShapeDtypeStruct((_T,), jnp.int32),
        ),
        mesh=plsc.VectorSubcoreMesh(
            core_axis_name="c", subcore_axis_name="s"),
        compiler_params=pltpu.CompilerParams(needs_layout_passes=False),
        scratch_types=[
            pltpu.VMEM((_SCB_TOK,), jnp.int32),
            pltpu.VMEM((_SCB_TOK,), jnp.int32),
            pltpu.VMEM((_SCB_TOK,), jnp.int32),
            pltpu.VMEM((_E,), jnp.int32),
            pltpu.VMEM((_SCB_TOK, _D), jnp.float32),
            pltpu.SemaphoreType.DMA,
        ],
    )
    return f(x, eid, rank, off)


def _scb_body(ypad_hbm, pos_hbm, out_hbm, posv_ref, rows_ref, sem):
    wid = lax.axis_index("s") * 2 + lax.axis_index("c")
    base = wid * _SCB_TOK
    pltpu.sync_copy(pos_hbm.at[pl.ds(base, _SCB_TOK)], posv_ref)
    cp = pltpu.make_async_copy(ypad_hbm.at[posv_ref], rows_ref, sem)
    cp.start()
    cp.wait()
    pltpu.sync_copy(rows_ref, out_hbm.at[pl.ds(base, _SCB_TOK)])


def _sc_combine(y_padded, pos):
    f = pl.kernel(
        _scb_body,
        out_type=jax.ShapeDtypeStruct((_T, _D), jnp.float32),
        mesh=plsc.VectorSubcoreMesh(
            core_axis_name="c", subcore_axis_name="s"),
        compiler_params=pltpu.CompilerParams(needs_layout_passes=False),
        scratch_types=[
            pltpu.VMEM((_SCB_TOK,), jnp.int32),
            pltpu.VMEM((_SCB_TOK, _D), jnp.float32),
            pltpu.SemaphoreType.DMA,
        ],
    )
    return f(y_padded, pos)


def _gemm_body(eot_ref, fetch_ref, chidx_ref, chexp_ref, chexists_ref,
               valid_ref, x_ref, wp_ref, wg_hbm, wu_hbm, wd_hbm, y_ref,
               wgb, wub, wdb, semg, semu, semd):
    s = pl.program_id(0)
    c = chidx_ref[s]
    b = jax.lax.rem(c, _NBUF)

    def start_fetch(slot, e):
        pltpu.make_async_copy(wg_hbm.at[e], wgb.at[slot], semg.at[slot]).start()
        pltpu.make_async_copy(wu_hbm.at[e], wub.at[slot], semu.at[slot]).start()
        pltpu.make_async_copy(wd_hbm.at[e], wdb.at[slot], semd.at[slot]).start()

    @pl.when(s == 0)
    def _():
        for c0 in range(_NBUF - 1):
            @pl.when(chexists_ref[c0] == 1)
            def _():
                start_fetch(c0, chexp_ref[c0])

    @pl.when(fetch_ref[s] == 1)
    def _():
        c2 = c + _NBUF - 1

        @pl.when(chexists_ref[c2] == 1)
        def _():
            start_fetch(jax.lax.rem(c2, _NBUF), chexp_ref[c2])

        e = eot_ref[s]
        pltpu.make_async_copy(wg_hbm.at[e], wgb.at[b], semg.at[b]).wait()
        pltpu.make_async_copy(wu_hbm.at[e], wub.at[b], semu.at[b]).wait()
        pltpu.make_async_copy(wd_hbm.at[e], wdb.at[b], semd.at[b]).wait()

    @pl.when(valid_ref[s] == 1)
    def _():
        x = x_ref[...].astype(jnp.bfloat16)
        g = jax.lax.dot_general(x, wgb[b].astype(jnp.bfloat16),
                                (((1,), (1,)), ((), ())),
                                preferred_element_type=jnp.float32)
        u = jax.lax.dot_general(x, wub[b].astype(jnp.bfloat16),
                                (((1,), (1,)), ((), ())),
                                preferred_element_type=jnp.float32)
        h = (g * jax.nn.sigmoid(g) * u).astype(jnp.bfloat16)
        y = jax.lax.dot_general(h, wdb[b].astype(jnp.bfloat16),
                                (((1,), (1,)), ((), ())),
                                preferred_element_type=jnp.float32)
        y_ref[...] = y * wp_ref[...]


def _grouped_gemm(eot, fetch, chidx, chexp, chexists, valid,
                  x_padded, w_padded, w_gate, w_up, w_down):
    grid_spec = pltpu.PrefetchScalarGridSpec(
        num_scalar_prefetch=6,
        grid=(_NTILES,),
        in_specs=[
            pl.BlockSpec((_BM, _D), lambda s, *_: (s, 0)),
            pl.BlockSpec((_BM, 1), lambda s, *_: (s, 0)),
            pl.BlockSpec(memory_space=pltpu.HBM),
            pl.BlockSpec(memory_space=pltpu.HBM),
            pl.BlockSpec(memory_space=pltpu.HBM),
        ],
        out_specs=pl.BlockSpec((_BM, _D), lambda s, *_: (s, 0)),
        scratch_shapes=[
            pltpu.VMEM((_NBUF, _DFF, _D), jnp.float32),
            pltpu.VMEM((_NBUF, _DFF, _D), jnp.float32),
            pltpu.VMEM((_NBUF, _D, _DFF), jnp.float32),
            pltpu.SemaphoreType.DMA((_NBUF,)),
            pltpu.SemaphoreType.DMA((_NBUF,)),
            pltpu.SemaphoreType.DMA((_NBUF,)),
        ],
    )
    return pl.pallas_call(
        _gemm_body,
        grid_spec=grid_spec,
        out_shape=jax.ShapeDtypeStruct((_TPAD, _D), jnp.float32),
        compiler_params=pltpu.CompilerParams(
            vmem_limit_bytes=100 * 1024 * 1024),
    )(eot, fetch, chidx, chexp, chexists, valid,
      x_padded, w_padded, w_gate, w_up, w_down)


def kernel(inputs, W_router, W_gate, W_up, W_down):
    x = inputs
    logits = _router_logits(x, W_router)

    eid, rank, w_den, cnt_tile = _sc_route(logits)
    w_tok = 1.0 / w_den

    counts = jnp.sum(cnt_tile, axis=0)
    padded = ((counts + _BM - 1) // _BM) * _BM
    base = jnp.cumsum(padded) - padded
    prior = jnp.cumsum(cnt_tile, axis=0) - cnt_tile
    off = (base[None, :] + prior).astype(jnp.int32)

    x_padded, pos = _sc_dispatch(x, eid, rank, off)
    s_idx = jnp.arange(_NTILES)
    tile_start = base // _BM
    tile_end = (base + padded) // _BM
    in_range = (s_idx[:, None] >= tile_start[None]) & (s_idx[:, None] < tile_end[None])
    eot = jnp.sum(jnp.arange(_E)[None] * in_range, axis=1).astype(jnp.int32)
    valid = jnp.any(in_range, axis=1).astype(jnp.int32)
    eot = jnp.where(valid == 1, eot, jnp.max(eot * valid)).astype(jnp.int32)
    change = jnp.concatenate([jnp.ones((1,), jnp.int32),
                              (eot[1:] != eot[:-1]).astype(jnp.int32)])
    chidx = (jnp.cumsum(change) - 1).astype(jnp.int32)
    nz = (counts > 0).astype(jnp.int32)
    crank = jnp.cumsum(nz) - nz
    c_idx = jnp.arange(_E + _NBUF)
    match = (c_idx[:, None] == crank[None, :]) & (nz[None, :] == 1)
    chexp = jnp.sum(jnp.where(match, jnp.arange(_E)[None, :], 0),
                    axis=1).astype(jnp.int32)
    chexists = (c_idx < jnp.sum(nz)).astype(jnp.int32)

    w_padded = jnp.zeros((_TPAD, 1), jnp.float32).at[pos, 0].set(w_tok)

    y_padded = _grouped_gemm(eot, change, chidx, chexp, chexists, valid,
                             x_padded, w_padded, W_gate, W_up, W_down)
    return _sc_combine(y_padded, pos)


# speedup vs baseline: 1.0377x; 1.0377x over previous
import jax
import jax.numpy as jnp
from jax import lax
from jax.experimental import pallas as pl
from jax.experimental.pallas import tpu as pltpu
from jax.experimental.pallas import tpu_sc as plsc

_E = 16
_D = 768
_DFF = 2048
_T = 2048
_BM = 128
_NTILES = 32
_TPAD = _NTILES * _BM
_NBUF = 3

_SCA_W = 16
_SCA_TOK = _T // _SCA_W
_SCB_W = 32
_SCB_TOK = _T // _SCB_W


def _router_body(x_ref, wr_ref, out_ref):
    out_ref[...] = jax.lax.dot_general(
        x_ref[...], wr_ref[...], (((1,), (1,)), ((), ())),
        preferred_element_type=jnp.float32)


def _router_logits(x, w_router):
    return pl.pallas_call(
        _router_body,
        out_shape=jax.ShapeDtypeStruct((_T, _E), jnp.float32),
    )(x, w_router)


def _sca1_body(logits_hbm, eid_hbm, rank_hbm, wden_hbm, cnt_hbm,
               lv_ref, eid_ref, rank_ref, wv_ref, cnt_ref):
    wid = lax.axis_index("s") * 2 + lax.axis_index("c")
    base = wid * _SCB_TOK
    pltpu.sync_copy(logits_hbm.at[pl.ds(base, _SCB_TOK)], lv_ref)

    lane = lax.iota(jnp.int32, 16)
    cnt = jnp.zeros((16,), jnp.int32)
    for g in range(_SCB_TOK // 16):
        acc_e = jnp.zeros((16,), jnp.int32)
        acc_r = jnp.zeros((16,), jnp.int32)
        acc_w = jnp.zeros((16,), jnp.float32)
        for j in range(16):
            lv = lv_ref[g * 16 + j, :]
            m = jnp.max(lv)
            e_splat = plsc.all_reduce_ffs(lv == m)
            wgt = jnp.sum(jnp.exp(lv - m))
            onehot = (lane == e_splat).astype(jnp.int32)
            rank = jnp.sum(cnt * onehot)
            cnt = cnt + onehot
            sel = lane == j
            acc_e = jnp.where(sel, e_splat, acc_e)
            acc_r = jnp.where(sel, rank, acc_r)
            acc_w = jnp.where(sel, wgt, acc_w)
        eid_ref[pl.ds(g * 16, 16)] = acc_e
        rank_ref[pl.ds(g * 16, 16)] = acc_r
        wv_ref[pl.ds(g * 16, 16)] = acc_w

    cnt_ref[...] = cnt
    pltpu.sync_copy(eid_ref, eid_hbm.at[pl.ds(base, _SCB_TOK)])
    pltpu.sync_copy(rank_ref, rank_hbm.at[pl.ds(base, _SCB_TOK)])
    pltpu.sync_copy(wv_ref, wden_hbm.at[pl.ds(base, _SCB_TOK)])
    pltpu.sync_copy(cnt_ref, cnt_hbm.at[wid])


def _sc_route(logits):
    f = pl.kernel(
        _sca1_body,
        out_type=(
            jax.ShapeDtypeStruct((_T,), jnp.int32),
            jax.ShapeDtypeStruct((_T,), jnp.int32),
            jax.ShapeDtypeStruct((_T,), jnp.float32),
            jax.ShapeDtypeStruct((_SCB_W, _E), jnp.int32),
        ),
        mesh=plsc.VectorSubcoreMesh(
            core_axis_name="c", subcore_axis_name="s"),
        compiler_params=pltpu.CompilerParams(needs_layout_passes=False),
        scratch_types=[
            pltpu.VMEM((_SCB_TOK, _E), jnp.float32),
            pltpu.VMEM((_SCB_TOK,), jnp.int32),
            pltpu.VMEM((_SCB_TOK,), jnp.int32),
            pltpu.VMEM((_SCB_TOK,), jnp.float32),
            pltpu.VMEM((_E,), jnp.int32),
        ],
    )
    return f(logits)


def _sca2_body(x_hbm, eid_hbm, rank_hbm, off_hbm,
               xpad_hbm, pos_hbm,
               eid_ref, rank_ref, posv_ref, off_ref, xrow_ref,
               sem):
    wid = lax.axis_index("s") * 2 + lax.axis_index("c")
    base = wid * _SCB_TOK
    pltpu.sync_copy(eid_hbm.at[pl.ds(base, _SCB_TOK)], eid_ref)
    pltpu.sync_copy(rank_hbm.at[pl.ds(base, _SCB_TOK)], rank_ref)
    pltpu.sync_copy(off_hbm.at[wid], off_ref)

    for g in range(_SCB_TOK // 16):
        ev = eid_ref[pl.ds(g * 16, 16)]
        offg = plsc.load_gather(off_ref, [ev])
        posv_ref[pl.ds(g * 16, 16)] = offg + rank_ref[pl.ds(g * 16, 16)]

    pltpu.sync_copy(x_hbm.at[pl.ds(base, _SCB_TOK)], xrow_ref)
    cp = pltpu.make_async_copy(xrow_ref, xpad_hbm.at[posv_ref], sem)
    cp.start()
    cp.wait()
    pltpu.sync_copy(posv_ref, pos_hbm.at[pl.ds(base, _SCB_TOK)])


def _sc_dispatch(x, eid, rank, off):
    f = pl.kernel(
        _sca2_body,
        out_type=(
            jax.ShapeDtypeStruct((_TPAD, _D), jnp.float32),
            jax.ShapeDtypeStruct((_T,), jnp.int32),
        ),
        mesh=plsc.VectorSubcoreMesh(
            core_axis_name="c", subcore_axis_name="s"),
        compiler_params=pltpu.CompilerParams(needs_layout_passes=False),
        scratch_types=[
            pltpu.VMEM((_SCB_TOK,), jnp.int32),
            pltpu.VMEM((_SCB_TOK,), jnp.int32),
            pltpu.VMEM((_SCB_TOK,), jnp.int32),
            pltpu.VMEM((_E,), jnp.int32),
            pltpu.VMEM((_SCB_TOK, _D), jnp.float32),
            pltpu.SemaphoreType.DMA,
        ],
    )
    return f(x, eid, rank, off)


def _scb_body(ypad_hbm, pos_hbm, out_hbm, posv_ref, rows_ref, sem):
    wid = lax.axis_index("s") * 2 + lax.axis_index("c")
    base = wid * _SCB_TOK
    pltpu.sync_copy(pos_hbm.at[pl.ds(base, _SCB_TOK)], posv_ref)
    cp = pltpu.make_async_copy(ypad_hbm.at[posv_ref], rows_ref, sem)
    cp.start()
    cp.wait()
    pltpu.sync_copy(rows_ref, out_hbm.at[pl.ds(base, _SCB_TOK)])


def _sc_combine(y_padded, pos):
    f = pl.kernel(
        _scb_body,
        out_type=jax.ShapeDtypeStruct((_T, _D), jnp.float32),
        mesh=plsc.VectorSubcoreMesh(
            core_axis_name="c", subcore_axis_name="s"),
        compiler_params=pltpu.CompilerParams(needs_layout_passes=False),
        scratch_types=[
            pltpu.VMEM((_SCB_TOK,), jnp.int32),
            pltpu.VMEM((_SCB_TOK, _D), jnp.float32),
            pltpu.SemaphoreType.DMA,
        ],
    )
    return f(y_padded, pos)


def _gemm_body(eot_ref, fetch_ref, chidx_ref, chexp_ref, chexists_ref,
               valid_ref, x_ref, wg_hbm, wu_hbm, wd_hbm, y_ref,
               wgb, wub, wdb, semg, semu, semd):
    s = pl.program_id(0)
    c = chidx_ref[s]
    b = jax.lax.rem(c, _NBUF)

    def start_fetch(slot, e):
        pltpu.make_async_copy(wg_hbm.at[e], wgb.at[slot], semg.at[slot]).start()
        pltpu.make_async_copy(wu_hbm.at[e], wub.at[slot], semu.at[slot]).start()
        pltpu.make_async_copy(wd_hbm.at[e], wdb.at[slot], semd.at[slot]).start()

    @pl.when(s == 0)
    def _():
        for c0 in range(_NBUF - 1):
            @pl.when(chexists_ref[c0] == 1)
            def _():
                start_fetch(c0, chexp_ref[c0])

    @pl.when(fetch_ref[s] == 1)
    def _():
        e = eot_ref[s]
        pltpu.make_async_copy(wg_hbm.at[e], wgb.at[b], semg.at[b]).wait()
        pltpu.make_async_copy(wu_hbm.at[e], wub.at[b], semu.at[b]).wait()
        pltpu.make_async_copy(wd_hbm.at[e], wdb.at[b], semd.at[b]).wait()
        c2 = c + _NBUF - 1

        @pl.when(chexists_ref[c2] == 1)
        def _():
            start_fetch(jax.lax.rem(c2, _NBUF), chexp_ref[c2])

    @pl.when(valid_ref[s] == 1)
    def _():
        x = x_ref[...].astype(jnp.bfloat16)
        g = jax.lax.dot_general(x, wgb[b].astype(jnp.bfloat16),
                                (((1,), (1,)), ((), ())),
                                preferred_element_type=jnp.float32)
        u = jax.lax.dot_general(x, wub[b].astype(jnp.bfloat16),
                                (((1,), (1,)), ((), ())),
                                preferred_element_type=jnp.float32)
        h = (g * jax.nn.sigmoid(g) * u).astype(jnp.bfloat16)
        y_ref[...] = jax.lax.dot_general(
            h, wdb[b].astype(jnp.bfloat16), (((1,), (1,)), ((), ())),
            preferred_element_type=jnp.float32)


def _grouped_gemm(eot, fetch, chidx, chexp, chexists, valid,
                  x_padded, w_gate, w_up, w_down):
    grid_spec = pltpu.PrefetchScalarGridSpec(
        num_scalar_prefetch=6,
        grid=(_NTILES,),
        in_specs=[
            pl.BlockSpec((_BM, _D), lambda s, *_: (s, 0)),
            pl.BlockSpec(memory_space=pltpu.HBM),
            pl.BlockSpec(memory_space=pltpu.HBM),
            pl.BlockSpec(memory_space=pltpu.HBM),
        ],
        out_specs=pl.BlockSpec((_BM, _D), lambda s, *_: (s, 0)),
        scratch_shapes=[
            pltpu.VMEM((_NBUF, _DFF, _D), jnp.float32),
            pltpu.VMEM((_NBUF, _DFF, _D), jnp.float32),
            pltpu.VMEM((_NBUF, _D, _DFF), jnp.float32),
            pltpu.SemaphoreType.DMA((_NBUF,)),
            pltpu.SemaphoreType.DMA((_NBUF,)),
            pltpu.SemaphoreType.DMA((_NBUF,)),
        ],
    )
    return pl.pallas_call(
        _gemm_body,
        grid_spec=grid_spec,
        out_shape=jax.ShapeDtypeStruct((_TPAD, _D), jnp.float32),
        compiler_params=pltpu.CompilerParams(
            vmem_limit_bytes=100 * 1024 * 1024),
    )(eot, fetch, chidx, chexp, chexists, valid,
      x_padded, w_gate, w_up, w_down)


def kernel(inputs, W_router, W_gate, W_up, W_down):
    x = inputs
    logits = _router_logits(x, W_router)

    eid, rank, w_den, cnt_tile = _sc_route(logits)
    w_tok = 1.0 / w_den

    counts = jnp.sum(cnt_tile, axis=0)
    padded = ((counts + _BM - 1) // _BM) * _BM
    base = jnp.cumsum(padded) - padded
    prior = jnp.cumsum(cnt_tile, axis=0) - cnt_tile
    off = (base[None, :] + prior).astype(jnp.int32)

    x_padded, pos = _sc_dispatch(x, eid, rank, off)
    s_idx = jnp.arange(_NTILES)
    tile_start = base // _BM
    tile_end = (base + padded) // _BM
    in_range = (s_idx[:, None] >= tile_start[None]) & (s_idx[:, None] < tile_end[None])
    eot = jnp.sum(jnp.arange(_E)[None] * in_range, axis=1).astype(jnp.int32)
    valid = jnp.any(in_range, axis=1).astype(jnp.int32)
    eot = jnp.where(valid == 1, eot, jnp.max(eot * valid)).astype(jnp.int32)
    change = jnp.concatenate([jnp.ones((1,), jnp.int32),
                              (eot[1:] != eot[:-1]).astype(jnp.int32)])
    chidx = (jnp.cumsum(change) - 1).astype(jnp.int32)
    nz = (counts > 0).astype(jnp.int32)
    crank = jnp.cumsum(nz) - nz
    c_idx = jnp.arange(_E + _NBUF)
    match = (c_idx[:, None] == crank[None, :]) & (nz[None, :] == 1)
    chexp = jnp.sum(jnp.where(match, jnp.arange(_E)[None, :], 0),
                    axis=1).astype(jnp.int32)
    chexists = (c_idx < jnp.sum(nz)).astype(jnp.int32)

    y_padded = _grouped_gemm(eot, change, chidx, chexp, chexists, valid,
                             x_padded, W_gate, W_up, W_down)
    return _sc_combine(y_padded, pos) * w_tok[:, None]


# speedup vs baseline: 1.0379x; 1.0002x over previous
import jax
import jax.numpy as jnp
from jax import lax
from jax.experimental import pallas as pl
from jax.experimental.pallas import tpu as pltpu
from jax.experimental.pallas import tpu_sc as plsc

_E = 16
_D = 768
_DFF = 2048
_T = 2048
_BM = 128
_NTILES = 32
_TPAD = _NTILES * _BM
_NBUF = 3

_SCB_W = 32
_SCB_TOK = _T // _SCB_W


def _router_body(x_ref, wr_ref, out_ref):
    out_ref[...] = jax.lax.dot_general(
        x_ref[...], wr_ref[...], (((1,), (1,)), ((), ())),
        preferred_element_type=jnp.float32)


def _router_logits(x, w_router):
    return pl.pallas_call(
        _router_body,
        out_shape=jax.ShapeDtypeStruct((_T, _E), jnp.float32),
    )(x, w_router)


def _sca1_body(logits_hbm, eid_hbm, rank_hbm, wden_hbm, cnt_hbm,
               lv_ref, eid_ref, rank_ref, wv_ref, cnt_ref):
    wid = lax.axis_index("s") * 2 + lax.axis_index("c")
    base = wid * _SCB_TOK
    pltpu.sync_copy(logits_hbm.at[pl.ds(base, _SCB_TOK)], lv_ref)

    lane = lax.iota(jnp.int32, 16)
    cnt = jnp.zeros((16,), jnp.int32)
    for g in range(_SCB_TOK // 16):
        acc_e = jnp.zeros((16,), jnp.int32)
        acc_r = jnp.zeros((16,), jnp.int32)
        acc_w = jnp.zeros((16,), jnp.float32)
        for j in range(16):
            lv = lv_ref[g * 16 + j, :]
            m = jnp.max(lv)
            e_splat = plsc.all_reduce_ffs(lv == m)
            wgt = jnp.sum(jnp.exp(lv - m))
            onehot = (lane == e_splat).astype(jnp.int32)
            rank = jnp.sum(cnt * onehot)
            cnt = cnt + onehot
            sel = lane == j
            acc_e = jnp.where(sel, e_splat, acc_e)
            acc_r = jnp.where(sel, rank, acc_r)
            acc_w = jnp.where(sel, wgt, acc_w)
        eid_ref[pl.ds(g * 16, 16)] = acc_e
        rank_ref[pl.ds(g * 16, 16)] = acc_r
        wv_ref[pl.ds(g * 16, 16)] = acc_w

    cnt_ref[...] = cnt
    pltpu.sync_copy(eid_ref, eid_hbm.at[pl.ds(base, _SCB_TOK)])
    pltpu.sync_copy(rank_ref, rank_hbm.at[pl.ds(base, _SCB_TOK)])
    pltpu.sync_copy(wv_ref, wden_hbm.at[pl.ds(base, _SCB_TOK)])
    pltpu.sync_copy(cnt_ref, cnt_hbm.at[wid])


def _sc_route(logits):
    f = pl.kernel(
        _sca1_body,
        out_type=(
            jax.ShapeDtypeStruct((_T,), jnp.int32),
            jax.ShapeDtypeStruct((_T,), jnp.int32),
            jax.ShapeDtypeStruct((_T,), jnp.float32),
            jax.ShapeDtypeStruct((_SCB_W, _E), jnp.int32),
        ),
        mesh=plsc.VectorSubcoreMesh(
            core_axis_name="c", subcore_axis_name="s"),
        compiler_params=pltpu.CompilerParams(needs_layout_passes=False),
        scratch_types=[
            pltpu.VMEM((_SCB_TOK, _E), jnp.float32),
            pltpu.VMEM((_SCB_TOK,), jnp.int32),
            pltpu.VMEM((_SCB_TOK,), jnp.int32),
            pltpu.VMEM((_SCB_TOK,), jnp.float32),
            pltpu.VMEM((_E,), jnp.int32),
        ],
    )
    return f(logits)


def _sca2_body(x_hbm, eid_hbm, rank_hbm, off_hbm,
               xpad_hbm, pos_hbm,
               eid_ref, rank_ref, posv_ref, off_ref, xrow_ref,
               sem):
    wid = lax.axis_index("s") * 2 + lax.axis_index("c")
    base = wid * _SCB_TOK
    pltpu.sync_copy(eid_hbm.at[pl.ds(base, _SCB_TOK)], eid_ref)
    pltpu.sync_copy(rank_hbm.at[pl.ds(base, _SCB_TOK)], rank_ref)
    pltpu.sync_copy(off_hbm.at[wid], off_ref)

    for g in range(_SCB_TOK // 16):
        ev = eid_ref[pl.ds(g * 16, 16)]
        offg = plsc.load_gather(off_ref, [ev])
        posv_ref[pl.ds(g * 16, 16)] = offg + rank_ref[pl.ds(g * 16, 16)]

    pltpu.sync_copy(x_hbm.at[pl.ds(base, _SCB_TOK)], xrow_ref)
    cp = pltpu.make_async_copy(xrow_ref, xpad_hbm.at[posv_ref], sem)
    cp.start()
    cp.wait()
    pltpu.sync_copy(posv_ref, pos_hbm.at[pl.ds(base, _SCB_TOK)])


def _sc_dispatch(x, eid, rank, off):
    f = pl.kernel(
        _sca2_body,
        out_type=(
            jax.ShapeDtypeStruct((_TPAD, _D), jnp.float32),
            jax.ShapeDtypeStruct((_T,), jnp.int32),
        ),
        mesh=plsc.VectorSubcoreMesh(
            core_axis_name="c", subcore_axis_name="s"),
        compiler_params=pltpu.CompilerParams(needs_layout_passes=False),
        scratch_types=[
            pltpu.VMEM((_SCB_TOK,), jnp.int32),
            pltpu.VMEM((_SCB_TOK,), jnp.int32),
            pltpu.VMEM((_SCB_TOK,), jnp.int32),
            pltpu.VMEM((_E,), jnp.int32),
            pltpu.VMEM((_SCB_TOK, _D), jnp.float32),
            pltpu.SemaphoreType.DMA,
        ],
    )
    return f(x, eid, rank, off)


def _scb_body(ypad_hbm, pos_hbm, out_hbm, posv_ref, rows_ref, sem):
    wid = lax.axis_index("s") * 2 + lax.axis_index("c")
    base = wid * _SCB_TOK
    pltpu.sync_copy(pos_hbm.at[pl.ds(base, _SCB_TOK)], posv_ref)
    cp = pltpu.make_async_copy(ypad_hbm.at[posv_ref], rows_ref, sem)
    cp.start()
    cp.wait()
    pltpu.sync_copy(rows_ref, out_hbm.at[pl.ds(base, _SCB_TOK)])


def _sc_combine(y_padded, pos):
    f = pl.kernel(
        _scb_body,
        out_type=jax.ShapeDtypeStruct((_T, _D), jnp.float32),
        mesh=plsc.VectorSubcoreMesh(
            core_axis_name="c", subcore_axis_name="s"),
        compiler_params=pltpu.CompilerParams(needs_layout_passes=False),
        scratch_types=[
            pltpu.VMEM((_SCB_TOK,), jnp.int32),
            pltpu.VMEM((_SCB_TOK, _D), jnp.float32),
            pltpu.SemaphoreType.DMA,
        ],
    )
    return f(y_padded, pos)


def _gemm_body(eot_ref, fetch_ref, chidx_ref, chexp_ref, chexists_ref,
               valid_ref, x_ref, wg_hbm, wu_hbm, wd_hbm, y_ref,
               wgb, wub, wdb, semg, semu, semd):
    s = pl.program_id(0)
    c = chidx_ref[s]
    b = jax.lax.rem(c, _NBUF)

    def start_fetch(slot, e):
        pltpu.make_async_copy(wg_hbm.at[e], wgb.at[slot], semg.at[slot]).start()
        pltpu.make_async_copy(wu_hbm.at[e], wub.at[slot], semu.at[slot]).start()
        pltpu.make_async_copy(wd_hbm.at[e], wdb.at[slot], semd.at[slot]).start()

    @pl.when(s == 0)
    def _():
        for c0 in range(_NBUF - 1):
            @pl.when(chexists_ref[c0] == 1)
            def _():
                start_fetch(c0, chexp_ref[c0])

    @pl.when(fetch_ref[s] == 1)
    def _():
        e = eot_ref[s]
        pltpu.make_async_copy(wg_hbm.at[e], wgb.at[b], semg.at[b]).wait()
        pltpu.make_async_copy(wu_hbm.at[e], wub.at[b], semu.at[b]).wait()
        pltpu.make_async_copy(wd_hbm.at[e], wdb.at[b], semd.at[b]).wait()
        c2 = c + _NBUF - 1

        @pl.when(chexists_ref[c2] == 1)
        def _():
            start_fetch(jax.lax.rem(c2, _NBUF), chexp_ref[c2])

    @pl.when(valid_ref[s] == 1)
    def _():
        x = x_ref[...].astype(jnp.bfloat16)
        g = jax.lax.dot_general(x, wgb[b].astype(jnp.bfloat16),
                                (((1,), (1,)), ((), ())),
                                preferred_element_type=jnp.float32)
        u = jax.lax.dot_general(x, wub[b].astype(jnp.bfloat16),
                                (((1,), (1,)), ((), ())),
                                preferred_element_type=jnp.float32)
        h = (g * jax.nn.sigmoid(g) * u).astype(jnp.bfloat16)
        y_ref[...] = jax.lax.dot_general(
            h, wdb[b].astype(jnp.bfloat16), (((1,), (1,)), ((), ())),
            preferred_element_type=jnp.float32)


def _grouped_gemm(eot, fetch, chidx, chexp, chexists, valid,
                  x_padded, w_gate, w_up, w_down):
    grid_spec = pltpu.PrefetchScalarGridSpec(
        num_scalar_prefetch=6,
        grid=(_NTILES,),
        in_specs=[
            pl.BlockSpec((_BM, _D), lambda s, *_: (s, 0)),
            pl.BlockSpec(memory_space=pltpu.HBM),
            pl.BlockSpec(memory_space=pltpu.HBM),
            pl.BlockSpec(memory_space=pltpu.HBM),
        ],
        out_specs=pl.BlockSpec((_BM, _D), lambda s, *_: (s, 0)),
        scratch_shapes=[
            pltpu.VMEM((_NBUF, _DFF, _D), jnp.float32),
            pltpu.VMEM((_NBUF, _DFF, _D), jnp.float32),
            pltpu.VMEM((_NBUF, _D, _DFF), jnp.float32),
            pltpu.SemaphoreType.DMA((_NBUF,)),
            pltpu.SemaphoreType.DMA((_NBUF,)),
            pltpu.SemaphoreType.DMA((_NBUF,)),
        ],
    )
    return pl.pallas_call(
        _gemm_body,
        grid_spec=grid_spec,
        out_shape=jax.ShapeDtypeStruct((_TPAD, _D), jnp.float32),
        compiler_params=pltpu.CompilerParams(
            vmem_limit_bytes=100 * 1024 * 1024),
    )(eot, fetch, chidx, chexp, chexists, valid,
      x_padded, w_gate, w_up, w_down)


def kernel(inputs, W_router, W_gate, W_up, W_down):
    x = inputs
    logits = _router_logits(x, W_router)

    eid, rank, w_den, cnt_tile = _sc_route(logits)
    w_tok = 1.0 / w_den

    counts = jnp.sum(cnt_tile, axis=0)
    padded = ((counts + _BM - 1) // _BM) * _BM
    base = jnp.cumsum(padded) - padded
    prior = jnp.cumsum(cnt_tile, axis=0) - cnt_tile
    off = (base[None, :] + prior).astype(jnp.int32)

    x_padded, pos = _sc_dispatch(x, eid, rank, off)
    s_idx = jnp.arange(_NTILES)
    tile_start = base // _BM
    tile_end = (base + padded) // _BM
    in_range = (s_idx[:, None] >= tile_start[None]) & (s_idx[:, None] < tile_end[None])
    eot = jnp.sum(jnp.arange(_E)[None] * in_range, axis=1).astype(jnp.int32)
    valid = jnp.any(in_range, axis=1).astype(jnp.int32)
    eot = jnp.where(valid == 1, eot, jnp.max(eot * valid)).astype(jnp.int32)
    change = jnp.concatenate([jnp.ones((1,), jnp.int32),
                              (eot[1:] != eot[:-1]).astype(jnp.int32)])
    chidx = (jnp.cumsum(change) - 1).astype(jnp.int32)
    nz = (counts > 0).astype(jnp.int32)
    crank = jnp.cumsum(nz) - nz
    c_idx = jnp.arange(_E + _NBUF)
    match = (c_idx[:, None] == crank[None, :]) & (nz[None, :] == 1)
    chexp = jnp.sum(jnp.where(match, jnp.arange(_E)[None, :], 0),
                    axis=1).astype(jnp.int32)
    chexists = (c_idx < jnp.sum(nz)).astype(jnp.int32)

    y_padded = _grouped_gemm(eot, change, chidx, chexp, chexists, valid,
                             x_padded, W_gate, W_up, W_down)
    return _sc_combine(y_padded, pos) * w_tok[:, None]
